# baseline (speedup 1.0000x reference)
import jax
import jax.numpy as jnp
from jax.experimental import pallas as pl
from jax.experimental.pallas import tpu as pltpu

_B = 4096
_C = 20000
_D = 128
_CORES = 2
_ROWS = _B // _CORES
_CHUNK = 128


def _center_loss_kernel(labels_ref, x_ref, centers_ref, out_ref, ta_ref, tb_ref):
    base = pl.program_id(0) * _ROWS

    def pair(o, acc):
        r = o * (2 * _CHUNK)
        for j in range(_CHUNK):
            ta_ref[j] = centers_ref[labels_ref[base + r + j]]
        for j in range(_CHUNK):
            tb_ref[j] = centers_ref[labels_ref[base + r + _CHUNK + j]]
        da = x_ref[pl.ds(r, _CHUNK), :, :] - ta_ref[...]
        db = x_ref[pl.ds(r + _CHUNK, _CHUNK), :, :] - tb_ref[...]
        return acc + jnp.sum(da * da, axis=0) + jnp.sum(db * db, axis=0)

    acc = jax.lax.fori_loop(
        0, _ROWS // (2 * _CHUNK), pair, jnp.zeros((1, _D), jnp.float32)
    )
    out_ref[0, 0, :] = acc[0]


@jax.jit
def kernel(x, labels, centers):
    labels32 = labels.astype(jnp.int32)
    x3 = x.reshape(_B, 1, _D)
    c3 = centers.reshape(_C, 1, _D)
    grid_spec = pltpu.PrefetchScalarGridSpec(
        num_scalar_prefetch=1,
        grid=(_CORES,),
        in_specs=[
            pl.BlockSpec((_ROWS, 1, _D), lambda i, lbl: (i, 0, 0)),
            pl.BlockSpec((_C, 1, _D), lambda i, lbl: (0, 0, 0)),
        ],
        out_specs=pl.BlockSpec((1, 1, _D), lambda i, lbl: (i, 0, 0)),
        scratch_shapes=[
            pltpu.VMEM((_CHUNK, 1, _D), jnp.float32),
            pltpu.VMEM((_CHUNK, 1, _D), jnp.float32),
        ],
    )
    partials = pl.pallas_call(
        _center_loss_kernel,
        grid_spec=grid_spec,
        out_shape=jax.ShapeDtypeStruct((_CORES, 1, _D), jnp.float32),
        compiler_params=pltpu.CompilerParams(
            dimension_semantics=("parallel",),
        ),
    )(labels32, x3, c3)
    return jnp.sum(partials) / jnp.float32(_B * _C)


# speedup vs baseline: 1.2401x; 1.2401x over previous
import jax
import jax.numpy as jnp
from jax.experimental import pallas as pl
from jax.experimental.pallas import tpu as pltpu

_B = 4096
_C = 20000
_D = 128
_CORES = 2
_ROWS = _B // _CORES
_UNROLL = 256


def _center_loss_kernel(labels_ref, x_ref, centers_ref, out_ref):
    base = pl.program_id(0) * _ROWS

    def body(o, accs):
        acc0, acc1 = accs
        r = o * _UNROLL
        for j in range(0, _UNROLL, 2):
            d0 = x_ref[r + j, 0] - centers_ref[labels_ref[base + r + j], 0]
            d1 = x_ref[r + j + 1, 0] - centers_ref[labels_ref[base + r + j + 1], 0]
            acc0 = acc0 + d0 * d0
            acc1 = acc1 + d1 * d1
        return (acc0, acc1)

    z = jnp.zeros((_D,), jnp.float32)
    acc0, acc1 = jax.lax.fori_loop(0, _ROWS // _UNROLL, body, (z, z))
    out_ref[0, 0, :] = acc0 + acc1


@jax.jit
def kernel(x, labels, centers):
    labels32 = labels.astype(jnp.int32)
    x3 = x.reshape(_B, 1, _D)
    c3 = centers.reshape(_C, 1, _D)
    grid_spec = pltpu.PrefetchScalarGridSpec(
        num_scalar_prefetch=1,
        grid=(_CORES,),
        in_specs=[
            pl.BlockSpec((_ROWS, 1, _D), lambda i, lbl: (i, 0, 0)),
            pl.BlockSpec((_C, 1, _D), lambda i, lbl: (0, 0, 0)),
        ],
        out_specs=pl.BlockSpec((1, 1, _D), lambda i, lbl: (i, 0, 0)),
    )
    partials = pl.pallas_call(
        _center_loss_kernel,
        grid_spec=grid_spec,
        out_shape=jax.ShapeDtypeStruct((_CORES, 1, _D), jnp.float32),
        compiler_params=pltpu.CompilerParams(
            dimension_semantics=("parallel",),
        ),
    )(labels32, x3, c3)
    return jnp.sum(partials) / jnp.float32(_B * _C)
